# Initial kernel scaffold; baseline (speedup 1.0000x reference)
import jax
import jax.numpy as jnp
from jax.experimental import pallas as pl


def kernel(s_xyz, xyz, s_points, nsample):
    raise NotImplementedError("write your pallas kernel here")


# speedup vs baseline: 17.5449x; 17.5449x over previous
import functools

import jax
import jax.numpy as jnp
from jax import lax
from jax.experimental import pallas as pl
from jax.experimental.pallas import tpu as pltpu
from jax.experimental.pallas import tpu_sc as plsc

B = 2
NK = 8192
NQ = 8192
K = 16
CF = 64
CT = 3 + CF
TW = 128
QB = 256

TOTALQ = B * NQ
TOTALR = TOTALQ * K


def _topk_body(q_ref, kt_ref, out_ref):
    b = pl.program_id(0)
    q = q_ref[0]
    kt = kt_ref[0]
    mm = jnp.dot(q, kt, preferred_element_type=jnp.float32)
    qn = jnp.sum(q * q, axis=1, keepdims=True)
    kn = jnp.sum(kt * kt, axis=0, keepdims=True)
    d = -2.0 * mm
    d = d + qn
    d = d + kn
    iota = lax.broadcasted_iota(jnp.int32, (QB, NK), 1)
    cols = []
    for _ in range(K):
        m = jnp.min(d, axis=1, keepdims=True)
        am = jnp.min(jnp.where(d == m, iota, NK), axis=1, keepdims=True)
        cols.append(am)
        d = jnp.where(iota == am, jnp.inf, d)
    idx = jnp.concatenate(cols, axis=1)
    out_ref[0] = idx + b * NK


def _topk(xyz, s_xyz_t):
    return pl.pallas_call(
        _topk_body,
        grid=(B, NQ // QB),
        in_specs=[
            pl.BlockSpec((1, QB, 3), lambda b, q: (b, q, 0)),
            pl.BlockSpec((1, 3, NK), lambda b, q: (b, 0, 0)),
        ],
        out_specs=pl.BlockSpec((1, QB, K), lambda b, q: (b, q, 0)),
        out_shape=jax.ShapeDtypeStruct((B, NQ, K), jnp.int32),
    )(xyz, s_xyz_t)


_NC, _NS = 2, 16
NW = _NC * _NS
RW = TOTALR // NW
GSUB = 128
NCHUNK = RW // GSUB


def _gather_body(tab_hbm, idx_hbm, out_hbm, idx_v, rows_v, sem):
    wid = lax.axis_index("s") * _NC + lax.axis_index("c")

    def chunk(c, carry):
        base = pl.multiple_of(wid * RW + c * GSUB, GSUB)
        pltpu.sync_copy(idx_hbm.at[pl.ds(base, GSUB)], idx_v)
        pltpu.async_copy(tab_hbm.at[idx_v], rows_v, sem).wait()
        pltpu.sync_copy(rows_v, out_hbm.at[pl.ds(base, GSUB)])
        return carry

    lax.fori_loop(0, NCHUNK, chunk, 0)


def _gather(tab, idx_flat):
    mesh = plsc.VectorSubcoreMesh(core_axis_name="c", subcore_axis_name="s")
    f = functools.partial(
        pl.kernel,
        mesh=mesh,
        out_type=jax.ShapeDtypeStruct((TOTALR, TW), jnp.float32),
        scratch_types=[
            pltpu.VMEM((GSUB,), jnp.int32),
            pltpu.VMEM((GSUB, TW), jnp.float32),
            pltpu.SemaphoreType.DMA,
        ],
    )(_gather_body)
    return f(tab, idx_flat)


QB3 = 256


def _finish_body(g_ref, q_ref, np_ref, gx_ref):
    g = g_ref[...]
    q = q_ref[...]
    full = g - q[:, None, :]
    np_ref[...] = full[:, :, :CT]
    gx_ref[...] = full[:, :, :3]


def _finish(gathered, qpad):
    return pl.pallas_call(
        _finish_body,
        grid=(TOTALQ // QB3,),
        in_specs=[
            pl.BlockSpec((QB3, K, TW), lambda i: (i, 0, 0)),
            pl.BlockSpec((QB3, TW), lambda i: (i, 0)),
        ],
        out_specs=[
            pl.BlockSpec((QB3, K, CT), lambda i: (i, 0, 0)),
            pl.BlockSpec((QB3, K, 3), lambda i: (i, 0, 0)),
        ],
        out_shape=[
            jax.ShapeDtypeStruct((TOTALQ, K, CT), jnp.float32),
            jax.ShapeDtypeStruct((TOTALQ, K, 3), jnp.float32),
        ],
    )(gathered, qpad)


def kernel(s_xyz, xyz, s_points, nsample):
    del nsample
    s_xyz = s_xyz.astype(jnp.float32)
    xyz = xyz.astype(jnp.float32)
    s_points = s_points.astype(jnp.float32)

    s_xyz_t = jnp.transpose(s_xyz, (0, 2, 1))
    idx = _topk(xyz, s_xyz_t)

    tab = jnp.pad(jnp.concatenate([s_xyz, s_points], axis=-1),
                  ((0, 0), (0, 0), (0, TW - CT))).reshape(B * NK, TW)
    qpad = jnp.pad(xyz.reshape(TOTALQ, 3), ((0, 0), (0, TW - 3)))
    idx_flat = idx.reshape(TOTALR)

    gathered = _gather(tab, idx_flat).reshape(TOTALQ, K, TW)
    npf, gxf = _finish(gathered, qpad)
    new_points = npf.reshape(B, NQ, K, CT)
    gxyzn = gxf.reshape(B, NQ, K, 3)
    return (new_points, gxyzn)


# speedup vs baseline: 18.7027x; 1.0660x over previous
import functools

import jax
import jax.numpy as jnp
from jax import lax
from jax.experimental import pallas as pl
from jax.experimental.pallas import tpu as pltpu
from jax.experimental.pallas import tpu_sc as plsc

B = 2
NK = 8192
NQ = 8192
K = 16
CF = 64
CT = 3 + CF
TW = 128
XW = 16
QB = 256

TOTALQ = B * NQ
TOTALR = TOTALQ * K


def _topk_body(q_ref, kt_ref, out_ref):
    b = pl.program_id(0)
    q = q_ref[0]
    kt = kt_ref[0]
    mm = jnp.dot(q, kt, preferred_element_type=jnp.float32)
    qn = jnp.sum(q * q, axis=1, keepdims=True)
    kn = jnp.sum(kt * kt, axis=0, keepdims=True)
    d = -2.0 * mm
    d = d + qn
    d = d + kn
    iota = lax.broadcasted_iota(jnp.int32, (QB, NK), 1)
    cols = []
    for _ in range(K):
        am = jnp.argmin(d, axis=1)[:, None]
        cols.append(am)
        d = jnp.where(iota == am, jnp.inf, d)
    idx = jnp.concatenate(cols, axis=1)
    out_ref[0] = idx + b * NK


def _topk(xyz, s_xyz_t):
    return pl.pallas_call(
        _topk_body,
        grid=(B, NQ // QB),
        in_specs=[
            pl.BlockSpec((1, QB, 3), lambda b, q: (b, q, 0)),
            pl.BlockSpec((1, 3, NK), lambda b, q: (b, 0, 0)),
        ],
        out_specs=pl.BlockSpec((1, QB, K), lambda b, q: (b, q, 0)),
        out_shape=jax.ShapeDtypeStruct((B, NQ, K), jnp.int32),
    )(xyz, s_xyz_t)


_NC, _NS = 2, 16
NW = _NC * _NS
RW = TOTALR // NW
GSUB = 128
NCHUNK = RW // GSUB


def _gather_body(tab_hbm, idx_hbm, out_hbm, idx_v, rows_v, sem):
    wid = lax.axis_index("s") * _NC + lax.axis_index("c")

    def chunk(c, carry):
        base = pl.multiple_of(wid * RW + c * GSUB, GSUB)
        pltpu.sync_copy(idx_hbm.at[pl.ds(base, GSUB)], idx_v)
        pltpu.async_copy(tab_hbm.at[idx_v], rows_v, sem).wait()
        pltpu.sync_copy(rows_v, out_hbm.at[pl.ds(base, GSUB)])
        return carry

    lax.fori_loop(0, NCHUNK, chunk, 0)


def _gather(tab, idx_flat):
    mesh = plsc.VectorSubcoreMesh(core_axis_name="c", subcore_axis_name="s")
    f = functools.partial(
        pl.kernel,
        mesh=mesh,
        out_type=jax.ShapeDtypeStruct((TOTALR, TW), jnp.float32),
        scratch_types=[
            pltpu.VMEM((GSUB,), jnp.int32),
            pltpu.VMEM((GSUB, TW), jnp.float32),
            pltpu.SemaphoreType.DMA,
        ],
    )(_gather_body)
    return f(tab, idx_flat)


QB3 = 256


def _finish_body(g_ref, q_ref, np_ref, gxn_ref):
    g = g_ref[...]
    q = q_ref[...]
    xn16 = g[:, :, :XW] - q[:, None, :]
    np_ref[...] = jnp.concatenate([xn16, g[:, :, XW:CT]], axis=-1)
    gxn_ref[...] = xn16[:, :, :3]


def _finish(gathered, qpad):
    return pl.pallas_call(
        _finish_body,
        grid=(TOTALQ // QB3,),
        in_specs=[
            pl.BlockSpec((QB3, K, TW), lambda i: (i, 0, 0)),
            pl.BlockSpec((QB3, XW), lambda i: (i, 0)),
        ],
        out_specs=[
            pl.BlockSpec((QB3, K, CT), lambda i: (i, 0, 0)),
            pl.BlockSpec((QB3, K, 3), lambda i: (i, 0, 0)),
        ],
        out_shape=[
            jax.ShapeDtypeStruct((TOTALQ, K, CT), jnp.float32),
            jax.ShapeDtypeStruct((TOTALQ, K, 3), jnp.float32),
        ],
    )(gathered, qpad)


def kernel(s_xyz, xyz, s_points, nsample):
    del nsample
    s_xyz = s_xyz.astype(jnp.float32)
    xyz = xyz.astype(jnp.float32)
    s_points = s_points.astype(jnp.float32)

    s_xyz_t = jnp.transpose(s_xyz, (0, 2, 1))
    idx = _topk(xyz, s_xyz_t)

    tab = jnp.pad(jnp.concatenate([s_xyz, s_points], axis=-1),
                  ((0, 0), (0, 0), (0, TW - CT))).reshape(B * NK, TW)
    qpad = jnp.pad(xyz.reshape(TOTALQ, 3), ((0, 0), (0, XW - 3)))
    idx_flat = idx.reshape(TOTALR)

    gathered = _gather(tab, idx_flat).reshape(TOTALQ, K, TW)
    npf, gxf = _finish(gathered, qpad)
    new_points = npf.reshape(B, NQ, K, CT)
    gxyzn = gxf.reshape(B, NQ, K, 3)
    return (new_points, gxyzn)
